# baseline (speedup 1.0000x reference)
import functools

import jax
import jax.numpy as jnp
import numpy as np
from jax import lax
from jax.experimental import pallas as pl
from jax.experimental.pallas import tpu as pltpu
from jax.experimental.pallas import tpu_sc as plsc

_F32 = jnp.float32
_I32 = jnp.int32

D = 64
B = 16384
NC, NS = 2, 16
NW = NC * NS
ROWS_PER_W = B // NW
CHUNK = 128
NCHUNK = ROWS_PER_W // CHUNK
GROUPS = CHUNK // 16
UNROLL = 4

_EPS2 = np.float32(1e-24)
_TINY = np.float32(1e-30)


def _rsqrt(x):
    i = plsc.bitcast(x, _I32)
    i = np.int32(0x5F3759DF) - (i >> 1)
    y = plsc.bitcast(i, _F32)
    for _ in range(3):
        y = y * (np.float32(1.5) - np.float32(0.5) * x * y * y)
    return y


def _compute_chunk(h_v, r_v, t_v, w_v, s_v):
    lanes = lax.iota(_I32, 16)
    zero = jnp.zeros((16,), _F32)

    def group_body(g, carry):
        rows = g * 16 + lanes

        def dim_body(j, acc):
            hh, tt, rr, ww, hw, tw, hr, tr, ht, wr = acc
            for u in range(UNROLL):
                col = jnp.full((16,), j * UNROLL + u, _I32)
                hd = plsc.load_gather(h_v, [rows, col])
                rd = plsc.load_gather(r_v, [rows, col])
                td = plsc.load_gather(t_v, [rows, col])
                wd = plsc.load_gather(w_v, [rows, col])
                hh = hh + hd * hd
                tt = tt + td * td
                rr = rr + rd * rd
                ww = ww + wd * wd
                hw = hw + hd * wd
                tw = tw + td * wd
                hr = hr + hd * rd
                tr = tr + td * rd
                ht = ht + hd * td
                wr = wr + wd * rd
            return (hh, tt, rr, ww, hw, tw, hr, tr, ht, wr)

        hh, tt, rr, ww, hw, tw, hr, tr, ht, wr = lax.fori_loop(
            0, D // UNROLL, dim_body, (zero,) * 10)

        s = _rsqrt(jnp.maximum(ww, _EPS2))
        a = hw * s
        b = tw * s
        p2 = jnp.maximum(hh - a * a, np.float32(0.0))
        q2 = jnp.maximum(tt - b * b, np.float32(0.0))
        p = _rsqrt(jnp.maximum(p2, _EPS2))
        q = _rsqrt(jnp.maximum(q2, _EPS2))
        hvr = hr - a * s * wr
        tvr = tr - b * s * wr
        hvtv = ht - a * b
        d2 = (p2 * p * p + rr + q2 * q * q
              + np.float32(2.0) * (p * hvr - p * q * hvtv - q * tvr))
        d2 = jnp.maximum(d2, np.float32(0.0))
        score = d2 * _rsqrt(jnp.maximum(d2, _TINY))
        s_v[pl.ds(g * 16, 16)] = score
        return carry

    lax.fori_loop(0, GROUPS, group_body, 0)


def _body(idx_h, idx_r, idx_t, ent, rel, nv, out,
          ih_v, ir_v, it_v, h_v, r_v, t_v, w_v, s_v, sem):
    wid = lax.axis_index("s") * NC + lax.axis_index("c")
    base = wid * ROWS_PER_W
    for c in range(NCHUNK):
        off = base + c * CHUNK
        pltpu.sync_copy(idx_h.at[pl.ds(off, CHUNK)], ih_v)
        pltpu.sync_copy(idx_r.at[pl.ds(off, CHUNK)], ir_v)
        pltpu.sync_copy(idx_t.at[pl.ds(off, CHUNK)], it_v)
        cps = [pltpu.async_copy(ent.at[ih_v], h_v, sem),
               pltpu.async_copy(rel.at[ir_v], r_v, sem),
               pltpu.async_copy(nv.at[ir_v], w_v, sem),
               pltpu.async_copy(ent.at[it_v], t_v, sem)]
        for cp in cps:
            cp.wait()
        _compute_chunk(h_v, r_v, t_v, w_v, s_v)
        pltpu.sync_copy(s_v, out.at[pl.ds(off, CHUNK)])


_transh = functools.partial(
    pl.kernel,
    mesh=plsc.VectorSubcoreMesh(core_axis_name="c", subcore_axis_name="s"),
    out_type=jax.ShapeDtypeStruct((B,), _F32),
    compiler_params=pltpu.CompilerParams(
        needs_layout_passes=False, use_tc_tiling_on_sc=False),
    scratch_types=[
        pltpu.VMEM((CHUNK,), _I32),
        pltpu.VMEM((CHUNK,), _I32),
        pltpu.VMEM((CHUNK,), _I32),
        pltpu.VMEM((CHUNK, D), _F32),
        pltpu.VMEM((CHUNK, D), _F32),
        pltpu.VMEM((CHUNK, D), _F32),
        pltpu.VMEM((CHUNK, D), _F32),
        pltpu.VMEM((CHUNK,), _F32),
        pltpu.SemaphoreType.DMA,
    ],
)(_body)


def kernel(sample, entity_embedding, translation_embedding, norm_vector):
    sample = sample.astype(_I32)
    idx_h = sample[:, 0]
    idx_r = sample[:, 1]
    idx_t = sample[:, 2]
    return _transh(idx_h, idx_r, idx_t,
                   entity_embedding, translation_embedding, norm_vector)


# speedup vs baseline: 1.2781x; 1.2781x over previous
import functools

import jax
import jax.numpy as jnp
import numpy as np
from jax import lax
from jax.experimental import pallas as pl
from jax.experimental.pallas import tpu as pltpu
from jax.experimental.pallas import tpu_sc as plsc

_F32 = jnp.float32
_I32 = jnp.int32

D = 64
B = 16384
NC, NS = 2, 16
NW = NC * NS
ROWS_PER_W = B // NW
CHUNK = 128
NCHUNK = ROWS_PER_W // CHUNK
GROUPS = CHUNK // 16
UNROLL = 4

_EPS2 = np.float32(1e-24)
_TINY = np.float32(1e-30)


def _rsqrt(x):
    i = plsc.bitcast(x, _I32)
    i = np.int32(0x5F3759DF) - (i >> 1)
    y = plsc.bitcast(i, _F32)
    for _ in range(3):
        y = y * (np.float32(1.5) - np.float32(0.5) * x * y * y)
    return y


def _extract_indices(slab_v, ih_v, ir_v, it_v):
    lanes = lax.iota(_I32, 16)

    def body(g, carry):
        rows = g * 16 + lanes
        for col, dst in ((0, ih_v), (1, ir_v), (2, it_v)):
            c = jnp.zeros((16,), _I32) + col
            dst[pl.ds(g * 16, 16)] = plsc.load_gather(slab_v, [rows, c])
        return carry

    lax.fori_loop(0, GROUPS, body, 0)


def _compute_chunk(c, h_v, r_v, t_v, w_v, s_v):
    lanes = lax.iota(_I32, 16)
    zero = jnp.zeros((16,), _F32)

    def group_body(g, carry):
        rows = g * 16 + lanes

        def dim_body(j, acc):
            hh, tt, rr, ww, hw, tw, hr, tr, ht, wr = acc
            for u in range(UNROLL):
                col = (lanes + (j * UNROLL + u)) & 63
                hd = plsc.load_gather(h_v, [rows, col])
                rd = plsc.load_gather(r_v, [rows, col])
                td = plsc.load_gather(t_v, [rows, col])
                wd = plsc.load_gather(w_v, [rows, col])
                hh = hh + hd * hd
                tt = tt + td * td
                rr = rr + rd * rd
                ww = ww + wd * wd
                hw = hw + hd * wd
                tw = tw + td * wd
                hr = hr + hd * rd
                tr = tr + td * rd
                ht = ht + hd * td
                wr = wr + wd * rd
            return (hh, tt, rr, ww, hw, tw, hr, tr, ht, wr)

        hh, tt, rr, ww, hw, tw, hr, tr, ht, wr = lax.fori_loop(
            0, D // UNROLL, dim_body, (zero,) * 10)

        s = _rsqrt(jnp.maximum(ww, _EPS2))
        a = hw * s
        b = tw * s
        p2 = jnp.maximum(hh - a * a, np.float32(0.0))
        q2 = jnp.maximum(tt - b * b, np.float32(0.0))
        p = _rsqrt(jnp.maximum(p2, _EPS2))
        q = _rsqrt(jnp.maximum(q2, _EPS2))
        hvr = hr - a * s * wr
        tvr = tr - b * s * wr
        hvtv = ht - a * b
        d2 = (p2 * p * p + rr + q2 * q * q
              + np.float32(2.0) * (p * hvr - p * q * hvtv - q * tvr))
        d2 = jnp.maximum(d2, np.float32(0.0))
        score = d2 * _rsqrt(jnp.maximum(d2, _TINY))
        s_v[pl.ds(c * CHUNK + g * 16, 16)] = score
        return carry

    lax.fori_loop(0, GROUPS, group_body, 0)


def _body(sample, ent, rel, nv, out,
          slab_v, ih_v, ir_v, it_v, h_v, r_v, t_v, w_v, s_v, sems):
    wid = lax.axis_index("s") * NC + lax.axis_index("c")
    base = wid * ROWS_PER_W

    def stage(c):
        d = c % 2
        pltpu.sync_copy(sample.at[pl.ds(base + c * CHUNK, CHUNK), :],
                        slab_v.at[d])
        _extract_indices(slab_v.at[d], ih_v.at[d], ir_v.at[d], it_v.at[d])
        return [pltpu.async_copy(ent.at[ih_v.at[d]], h_v.at[d], sems.at[d]),
                pltpu.async_copy(rel.at[ir_v.at[d]], r_v.at[d], sems.at[d]),
                pltpu.async_copy(nv.at[ir_v.at[d]], w_v.at[d], sems.at[d]),
                pltpu.async_copy(ent.at[it_v.at[d]], t_v.at[d], sems.at[d])]

    inflight = stage(0)
    for c in range(NCHUNK):
        nxt = stage(c + 1) if c + 1 < NCHUNK else None
        for cp in inflight:
            cp.wait()
        d = c % 2
        _compute_chunk(c, h_v.at[d], r_v.at[d], t_v.at[d], w_v.at[d], s_v)
        inflight = nxt
    pltpu.sync_copy(s_v, out.at[pl.ds(base, ROWS_PER_W)])


_transh = functools.partial(
    pl.kernel,
    mesh=plsc.VectorSubcoreMesh(core_axis_name="c", subcore_axis_name="s"),
    out_type=jax.ShapeDtypeStruct((B,), _F32),
    compiler_params=pltpu.CompilerParams(
        needs_layout_passes=False, use_tc_tiling_on_sc=False),
    scratch_types=[
        pltpu.VMEM((2, CHUNK, 3), _I32),
        pltpu.VMEM((2, CHUNK), _I32),
        pltpu.VMEM((2, CHUNK), _I32),
        pltpu.VMEM((2, CHUNK), _I32),
        pltpu.VMEM((2, CHUNK, D), _F32),
        pltpu.VMEM((2, CHUNK, D), _F32),
        pltpu.VMEM((2, CHUNK, D), _F32),
        pltpu.VMEM((2, CHUNK, D), _F32),
        pltpu.VMEM((ROWS_PER_W,), _F32),
        pltpu.SemaphoreType.DMA((2,)),
    ],
)(_body)


def kernel(sample, entity_embedding, translation_embedding, norm_vector):
    return _transh(sample.astype(_I32), entity_embedding,
                   translation_embedding, norm_vector)
